# baseline (device time: 49019 ns/iter reference)
import jax
import jax.numpy as jnp
from jax import lax
from jax.experimental import pallas as pl
from jax.experimental.pallas import tpu as pltpu

N_DEV = 4
BLK = 64


def kernel(x, Wq, K_ext, V_ext, Wo):
    B, S, D = x.shape
    _, _, Hq, Dh = K_ext.shape
    HD = Hq * Dh
    n_blk = S // BLK
    Sh = S // 2

    k2 = K_ext.reshape(B, S, HD).astype(jnp.bfloat16)
    v2 = V_ext.reshape(B, S, HD).astype(jnp.bfloat16)

    def body(x_ref, wq_ref, k_ref, v_ref, wo_ref, out_ref,
             kv_all, q_ref, acc_ref, lsum_ref, ctx_ref, out_stage,
             send_sems, recv_sems, out_sems):
        my = lax.axis_index("i")
        left = lax.rem(my + N_DEV - 1, N_DEV)
        right = lax.rem(my + 1, N_DEV)

        barrier_sem = pltpu.get_barrier_semaphore()
        for nbr in (left, right):
            pl.semaphore_signal(
                barrier_sem, inc=1,
                device_id=(nbr,), device_id_type=pl.DeviceIdType.MESH,
            )
        pl.semaphore_wait(barrier_sem, 2)

        def mk(src, dst, s, dev):
            return pltpu.make_async_remote_copy(
                src_ref=src, dst_ref=dst,
                send_sem=send_sems.at[s], recv_sem=recv_sems.at[s],
                device_id=(dev,), device_id_type=pl.DeviceIdType.MESH,
            )

        aK_r = [mk(k_ref.at[b], kv_all.at[1, b, :, pl.ds(0, HD)], b, right)
                for b in range(B)]
        aV_r = [mk(v_ref.at[b], kv_all.at[1, b, :, pl.ds(HD, HD)], 2 + b,
                   right) for b in range(B)]
        aK_l = [mk(k_ref.at[b], kv_all.at[2, b, :, pl.ds(0, HD)], 4 + b, left)
                for b in range(B)]
        aV_l = [mk(v_ref.at[b], kv_all.at[2, b, :, pl.ds(HD, HD)], 6 + b,
                   left) for b in range(B)]
        b_r = [mk(kv_all.at[1, 0, pl.ds(i * Sh, Sh)],
                  kv_all.at[3, 0, pl.ds(i * Sh, Sh)], 8 + i, right)
               for i in range(2)]
        b_l = [mk(kv_all.at[2, 1, pl.ds(i * Sh, Sh)],
                  kv_all.at[3, 1, pl.ds(i * Sh, Sh)], 10 + i, left)
               for i in range(2)]

        aK_r[0].start()
        aV_r[0].start()
        aK_l[1].start()
        aV_l[1].start()
        aK_r[1].start()
        aV_r[1].start()
        aK_l[0].start()
        aV_l[0].start()

        wqb = wq_ref[...].astype(jnp.bfloat16)
        for b in range(B):
            q_ref[b] = (jnp.dot(
                x_ref[b].astype(jnp.bfloat16), wqb,
                preferred_element_type=jnp.float32,
            ) * 0.125).astype(jnp.bfloat16)

        tri = (lax.broadcasted_iota(jnp.int32, (S, S), 1) // BLK) <= (
            lax.broadcasted_iota(jnp.int32, (S, S), 0) // BLK)

        def consume(d, srck, srcv, voff, b, masked, r0=0, nr=S):
            ones_col = jnp.ones((nr, Dh), jnp.bfloat16)
            for h in range(Hq):
                qh = q_ref[b, :, h * Dh:(h + 1) * Dh]
                k_d = srck[b, r0:r0 + nr, h * Dh:(h + 1) * Dh]
                s_d = lax.dot_general(
                    qh, k_d, (((1,), (1,)), ((), ())),
                    preferred_element_type=jnp.float32,
                )
                w = jnp.exp(s_d)
                if masked:
                    w = jnp.where(tri, w, 0.0)
                wb = w.astype(jnp.bfloat16)
                v_d = srcv[b, r0:r0 + nr, voff + h * Dh:voff + (h + 1) * Dh]
                v1 = jnp.concatenate([v_d, ones_col], axis=1)
                pv1 = jnp.dot(wb, v1, preferred_element_type=jnp.float32)
                pv = pv1[:, :Dh]
                wsum = pv1[:, Dh:Dh + 1]
                if d == 0:
                    acc_ref[b, h] = pv
                    lsum_ref[b, :, h:h + 1] = wsum
                else:
                    acc_ref[b, h] = acc_ref[b, h] + pv
                    lsum_ref[b, :, h:h + 1] = lsum_ref[b, :, h:h + 1] + wsum

        def consume_side(d, origin, b, r0=0, nr=S):
            @pl.when(origin < my)
            def _():
                consume(d, kv_all.at[d], kv_all.at[d], HD, b,
                        masked=False, r0=r0, nr=nr)

        def finalize(b):
            for h in range(Hq):
                ctx_ref[:, h * Dh:(h + 1) * Dh] = (
                    acc_ref[b, h] / lsum_ref[b, :, h:h + 1]
                ).astype(jnp.bfloat16)
            out_stage[b] = jnp.dot(
                ctx_ref[...], wo_ref[...].astype(jnp.bfloat16),
                preferred_element_type=jnp.float32,
            ).astype(jnp.bfloat16)
            cp = pltpu.make_async_copy(
                out_stage.at[b], out_ref.at[b], out_sems.at[b])
            cp.start()
            return cp

        consume(0, k_ref, v_ref, 0, 0, masked=True)
        consume(0, k_ref, v_ref, 0, 1, masked=True)

        far = lax.rem(my + 2, N_DEV)

        aK_r[0].wait_recv()
        aV_r[0].wait_recv()
        b_r[0].start()
        b_r[1].start()
        consume_side(1, left, 0)
        aK_l[1].wait_recv()
        aV_l[1].wait_recv()
        b_l[0].start()
        b_l[1].start()
        consume_side(2, right, 1)

        aK_r[1].wait_recv()
        aV_r[1].wait_recv()
        consume_side(1, left, 1)
        aK_l[0].wait_recv()
        aV_l[0].wait_recv()
        consume_side(2, right, 0)

        b_r[0].wait_recv()
        consume_side(3, far, 0, 0, Sh)
        b_l[0].wait_recv()
        consume_side(3, far, 1, 0, Sh)
        b_r[1].wait_recv()
        consume_side(3, far, 0, Sh, Sh)
        cp0 = finalize(0)
        b_l[1].wait_recv()
        consume_side(3, far, 1, Sh, Sh)
        cp1 = finalize(1)
        cp0.wait()
        cp1.wait()

        for r in (*aK_r, *aV_r, *aK_l, *aV_l, *b_r, *b_l):
            r.wait_send()

    return pl.pallas_call(
        body,
        out_shape=jax.ShapeDtypeStruct((B, S, D), jnp.bfloat16),
        in_specs=[pl.BlockSpec(memory_space=pltpu.VMEM)] * 5,
        out_specs=pl.BlockSpec(memory_space=pl.ANY),
        scratch_shapes=[
            pltpu.VMEM((N_DEV, B, S, 2 * HD), jnp.bfloat16),
            pltpu.VMEM((B, S, HD), jnp.bfloat16),
            pltpu.VMEM((B, Hq, S, Dh), jnp.float32),
            pltpu.VMEM((B, S, Hq), jnp.float32),
            pltpu.VMEM((S, HD), jnp.bfloat16),
            pltpu.VMEM((B, S, D), jnp.bfloat16),
            pltpu.SemaphoreType.DMA((12,)),
            pltpu.SemaphoreType.DMA((12,)),
            pltpu.SemaphoreType.DMA((B,)),
        ],
        compiler_params=pltpu.CompilerParams(collective_id=0),
    )(x, Wq, k2, v2, Wo)


# device time: 48899 ns/iter; 1.0025x vs baseline; 1.0025x over previous
import jax
import jax.numpy as jnp
from jax import lax
from jax.experimental import pallas as pl
from jax.experimental.pallas import tpu as pltpu

N_DEV = 4
BLK = 64


def kernel(x, Wq, K_ext, V_ext, Wo):
    B, S, D = x.shape
    _, _, Hq, Dh = K_ext.shape
    HD = Hq * Dh
    n_blk = S // BLK
    Sh = S // 2

    k2 = K_ext.reshape(B, S, HD)
    v2 = V_ext.reshape(B, S, HD)

    def body(x_ref, wq_ref, k_ref, v_ref, wo_ref, out_ref,
             kv_own, kv_all, q_ref, acc_ref, lsum_ref, ctx_ref, out_stage,
             send_sems, recv_sems, out_sems):
        my = lax.axis_index("i")
        left = lax.rem(my + N_DEV - 1, N_DEV)
        right = lax.rem(my + 1, N_DEV)

        barrier_sem = pltpu.get_barrier_semaphore()
        for nbr in (left, right):
            pl.semaphore_signal(
                barrier_sem, inc=1,
                device_id=(nbr,), device_id_type=pl.DeviceIdType.MESH,
            )
        pl.semaphore_wait(barrier_sem, 2)

        def mk(src, dst, s, dev):
            return pltpu.make_async_remote_copy(
                src_ref=src, dst_ref=dst,
                send_sem=send_sems.at[s], recv_sem=recv_sems.at[s],
                device_id=(dev,), device_id_type=pl.DeviceIdType.MESH,
            )

        a_r = [mk(kv_own.at[b], kv_all.at[1, b], b, right) for b in range(B)]
        a_l = [mk(kv_own.at[b], kv_all.at[2, b], 2 + b, left)
               for b in range(B)]
        b_r = [mk(kv_all.at[1, 0, pl.ds(i * Sh, Sh)],
                  kv_all.at[3, 0, pl.ds(i * Sh, Sh)], 4 + i, right)
               for i in range(2)]
        b_l = [mk(kv_all.at[2, 1, pl.ds(i * Sh, Sh)],
                  kv_all.at[3, 1, pl.ds(i * Sh, Sh)], 6 + i, left)
               for i in range(2)]

        kv_own[0, :, :HD] = k_ref[0].astype(jnp.bfloat16)
        kv_own[0, :, HD:] = v_ref[0].astype(jnp.bfloat16)
        a_r[0].start()
        kv_own[1, :, :HD] = k_ref[1].astype(jnp.bfloat16)
        kv_own[1, :, HD:] = v_ref[1].astype(jnp.bfloat16)
        a_l[1].start()
        a_r[1].start()
        a_l[0].start()

        wqb = wq_ref[...].astype(jnp.bfloat16)
        for b in range(B):
            q_ref[b] = (jnp.dot(
                x_ref[b].astype(jnp.bfloat16), wqb,
                preferred_element_type=jnp.float32,
            ) * 0.125).astype(jnp.bfloat16)

        tri = (lax.broadcasted_iota(jnp.int32, (S, S), 1) // BLK) <= (
            lax.broadcasted_iota(jnp.int32, (S, S), 0) // BLK)

        def consume(d, src, b, masked, r0=0, nr=S):
            ones_col = jnp.ones((nr, Dh), jnp.bfloat16)
            for h in range(Hq):
                qh = q_ref[b, :, h * Dh:(h + 1) * Dh]
                k_d = src[b, r0:r0 + nr, h * Dh:(h + 1) * Dh]
                s_d = lax.dot_general(
                    qh, k_d, (((1,), (1,)), ((), ())),
                    preferred_element_type=jnp.float32,
                )
                w = jnp.exp(s_d)
                if masked:
                    w = jnp.where(tri, w, 0.0)
                wb = w.astype(jnp.bfloat16)
                v_d = src[b, r0:r0 + nr, HD + h * Dh:HD + (h + 1) * Dh]
                v1 = jnp.concatenate([v_d, ones_col], axis=1)
                pv1 = jnp.dot(wb, v1, preferred_element_type=jnp.float32)
                pv = pv1[:, :Dh]
                wsum = pv1[:, Dh:Dh + 1]
                if d == 0:
                    acc_ref[b, h] = pv
                    lsum_ref[b, :, h:h + 1] = wsum
                else:
                    acc_ref[b, h] = acc_ref[b, h] + pv
                    lsum_ref[b, :, h:h + 1] = lsum_ref[b, :, h:h + 1] + wsum

        def consume_side(d, origin, b, r0=0, nr=S):
            @pl.when(origin < my)
            def _():
                consume(d, kv_all.at[d], b, masked=False, r0=r0, nr=nr)

        def finalize(b):
            for h in range(Hq):
                ctx_ref[:, h * Dh:(h + 1) * Dh] = (
                    acc_ref[b, h] / lsum_ref[b, :, h:h + 1]
                ).astype(jnp.bfloat16)
            out_stage[b] = jnp.dot(
                ctx_ref[...], wo_ref[...].astype(jnp.bfloat16),
                preferred_element_type=jnp.float32,
            ).astype(jnp.bfloat16)
            cp = pltpu.make_async_copy(
                out_stage.at[b], out_ref.at[b], out_sems.at[b])
            cp.start()
            return cp

        consume(0, kv_own, 0, masked=True)
        consume(0, kv_own, 1, masked=True)

        far = lax.rem(my + 2, N_DEV)

        a_r[0].wait_recv()
        b_r[0].start()
        b_r[1].start()
        consume_side(1, left, 0)
        a_l[1].wait_recv()
        b_l[0].start()
        b_l[1].start()
        consume_side(2, right, 1)

        a_r[1].wait_recv()
        consume_side(1, left, 1)
        a_l[0].wait_recv()
        consume_side(2, right, 0)

        b_r[0].wait_recv()
        consume_side(3, far, 0, 0, Sh)
        b_l[0].wait_recv()
        consume_side(3, far, 1, 0, Sh)
        b_r[1].wait_recv()
        consume_side(3, far, 0, Sh, Sh)
        cp0 = finalize(0)
        b_l[1].wait_recv()
        consume_side(3, far, 1, Sh, Sh)
        cp1 = finalize(1)
        cp0.wait()
        cp1.wait()

        for r in (*a_r, *a_l, *b_r, *b_l):
            r.wait_send()

    return pl.pallas_call(
        body,
        out_shape=jax.ShapeDtypeStruct((B, S, D), jnp.bfloat16),
        in_specs=[pl.BlockSpec(memory_space=pltpu.VMEM)] * 5,
        out_specs=pl.BlockSpec(memory_space=pl.ANY),
        scratch_shapes=[
            pltpu.VMEM((B, S, 2 * HD), jnp.bfloat16),
            pltpu.VMEM((N_DEV, B, S, 2 * HD), jnp.bfloat16),
            pltpu.VMEM((B, S, HD), jnp.bfloat16),
            pltpu.VMEM((B, Hq, S, Dh), jnp.float32),
            pltpu.VMEM((B, S, Hq), jnp.float32),
            pltpu.VMEM((S, HD), jnp.bfloat16),
            pltpu.VMEM((B, S, D), jnp.bfloat16),
            pltpu.SemaphoreType.DMA((8,)),
            pltpu.SemaphoreType.DMA((8,)),
            pltpu.SemaphoreType.DMA((B,)),
        ],
        compiler_params=pltpu.CompilerParams(collective_id=0),
    )(x, Wq, k2, v2, Wo)
